# baseline (speedup 1.0000x reference)
import functools

import jax
import jax.numpy as jnp
from jax import lax
from jax.experimental import pallas as pl
from jax.experimental.pallas import tpu as pltpu
from jax.experimental.pallas import tpu_sc as plsc

_UNROLL = 16
_HEAD = 2560


def _chunk_pattern(per_w):
    if per_w > 4 * _HEAD:
        mid = (per_w - 2 * _HEAD) // 2
        return (_HEAD, mid, per_w - 2 * _HEAD - mid, _HEAD)
    return (per_w,)


def _quantize_body(pattern, K, x_hbm, c_hbm, o_hbm, *scratch):
    ng = len(pattern)
    xvs = scratch[0:ng]
    ovs = scratch[ng:2 * ng]
    cv = scratch[2 * ng]
    isems = scratch[2 * ng + 1:2 * ng + 1 + ng]
    osems = scratch[2 * ng + 1 + ng:2 * ng + 1 + 2 * ng]

    nc = lax.axis_size("c")
    wid = lax.axis_index("s") * nc + lax.axis_index("c")
    per_w = sum(pattern)
    base = wid * per_w
    pltpu.sync_copy(c_hbm, cv)

    offs = [sum(pattern[:g]) for g in range(ng)]
    in_h = [
        pltpu.async_copy(x_hbm.at[pl.ds(base + offs[g], pattern[g])],
                         xvs[g], isems[g])
        for g in range(ng)
    ]

    cvec = cv[pl.ds(0, 16)]
    ctop = cv[pl.ds(K - 16, 16)]
    c0 = jnp.full((16,), cvec[0], jnp.float32)
    cK = jnp.full((16,), ctop[15], jnp.float32)
    inv_step = jnp.full((16,), K - 1, jnp.float32) / (cK - c0)
    bias = 0.5 - c0 * inv_step
    lo = jnp.full((16,), 0.5, jnp.float32)
    hi = jnp.full((16,), K - 0.5, jnp.float32)

    out_h = []
    for g in range(ng):
        in_h[g].wait()
        xv, ov = xvs[g], ovs[g]

        @plsc.parallel_loop(0, pattern[g], step=16, unroll=_UNROLL)
        def _vec(off, xv=xv, ov=ov):
            v = xv[pl.ds(off, 16)]
            u = jnp.minimum(jnp.maximum(v * inv_step + bias, lo), hi)
            ov[pl.ds(off, 16)] = plsc.load_gather(cv, [u.astype(jnp.int32)])

        out_h.append(pltpu.async_copy(
            ov, o_hbm.at[pl.ds(base + offs[g], pattern[g])], osems[g]))
    for h in out_h:
        h.wait()


def kernel(x, center):
    K = center.shape[0]
    n = x.size
    info = plsc.get_sparse_core_info()
    nw = info.num_cores * info.num_subcores
    grain = 16 * _UNROLL * 2
    n_pad = ((n + nw * grain - 1) // (nw * grain)) * (nw * grain)
    per_w = n_pad // nw
    pattern = _chunk_pattern(per_w)

    if x.ndim == 4 and x.shape[0] % 8 == 0 and x.shape[1] % 128 == 0:
        d0, d1, d2, d3 = x.shape
        flat = (
            x.transpose(2, 3, 0, 1)
            .reshape(d2, d3, d0 // 8, 8, d1 // 128, 128)
            .transpose(0, 1, 2, 4, 3, 5)
            .reshape(-1)
        )
        unscramble = lambda o: (
            o.reshape(d2, d3, d0 // 8, d1 // 128, 8, 128)
            .transpose(0, 1, 2, 4, 3, 5)
            .reshape(d2, d3, d0, d1)
            .transpose(2, 3, 0, 1)
        )
    else:
        flat = x.reshape(-1)
        unscramble = lambda o: o.reshape(x.shape)
    if n_pad != n:
        flat = jnp.pad(flat, (0, n_pad - n))

    mesh = plsc.VectorSubcoreMesh(core_axis_name="c", subcore_axis_name="s")
    body = functools.partial(_quantize_body, pattern, K)
    scratch = (
        [pltpu.VMEM((sz,), jnp.float32) for sz in pattern]
        + [pltpu.VMEM((sz,), jnp.float32) for sz in pattern]
        + [pltpu.VMEM((K,), jnp.float32)]
        + [pltpu.SemaphoreType.DMA] * (2 * len(pattern))
    )
    out = pl.kernel(
        body,
        out_type=jax.ShapeDtypeStruct((n_pad,), jnp.float32),
        mesh=mesh,
        scratch_types=scratch,
        compiler_params=pltpu.CompilerParams(needs_layout_passes=False),
    )(flat, center)
    return unscramble(out[:n])


# speedup vs baseline: 1.0092x; 1.0092x over previous
import functools

import jax
import jax.numpy as jnp
from jax import lax
from jax.experimental import pallas as pl
from jax.experimental.pallas import tpu as pltpu
from jax.experimental.pallas import tpu_sc as plsc

_UNROLL = 16
_HEAD = 2560


def _chunk_pattern(per_w):
    if per_w % 2 == 0 and (per_w // 2) % 256 == 0:
        return (per_w // 2, per_w // 2)
    return (per_w,)


def _quantize_body(pattern, K, x_hbm, c_hbm, o_hbm, *scratch):
    ng = len(pattern)
    xvs = scratch[0:ng]
    ovs = scratch[ng:2 * ng]
    cv = scratch[2 * ng]
    isems = scratch[2 * ng + 1:2 * ng + 1 + ng]
    osems = scratch[2 * ng + 1 + ng:2 * ng + 1 + 2 * ng]

    nc = lax.axis_size("c")
    wid = lax.axis_index("s") * nc + lax.axis_index("c")
    per_w = sum(pattern)
    base = wid * per_w
    pltpu.sync_copy(c_hbm, cv)

    offs = [sum(pattern[:g]) for g in range(ng)]
    in_h = [
        pltpu.async_copy(x_hbm.at[pl.ds(base + offs[g], pattern[g])],
                         xvs[g], isems[g])
        for g in range(ng)
    ]

    cvec = cv[pl.ds(0, 16)]
    ctop = cv[pl.ds(K - 16, 16)]
    c0 = jnp.full((16,), cvec[0], jnp.float32)
    cK = jnp.full((16,), ctop[15], jnp.float32)
    inv_step = jnp.full((16,), K - 1, jnp.float32) / (cK - c0)
    bias = 0.5 - c0 * inv_step
    lo = jnp.full((16,), 0.5, jnp.float32)
    hi = jnp.full((16,), K - 0.5, jnp.float32)

    out_h = []
    for g in range(ng):
        in_h[g].wait()
        xv, ov = xvs[g], ovs[g]

        @plsc.parallel_loop(0, pattern[g], step=16, unroll=_UNROLL)
        def _vec(off, xv=xv, ov=ov):
            v = xv[pl.ds(off, 16)]
            u = jnp.minimum(jnp.maximum(v * inv_step + bias, lo), hi)
            ov[pl.ds(off, 16)] = plsc.load_gather(cv, [u.astype(jnp.int32)])

        out_h.append(pltpu.async_copy(
            ov, o_hbm.at[pl.ds(base + offs[g], pattern[g])], osems[g]))
    for h in out_h:
        h.wait()


def kernel(x, center):
    K = center.shape[0]
    n = x.size
    info = plsc.get_sparse_core_info()
    nw = info.num_cores * info.num_subcores
    grain = 16 * _UNROLL * 2
    n_pad = ((n + nw * grain - 1) // (nw * grain)) * (nw * grain)
    per_w = n_pad // nw
    pattern = _chunk_pattern(per_w)

    if x.ndim == 4 and x.shape[0] % 8 == 0 and x.shape[1] % 128 == 0:
        d0, d1, d2, d3 = x.shape
        flat = (
            x.transpose(2, 3, 0, 1)
            .reshape(d2, d3, d0 // 8, 8, d1 // 128, 128)
            .transpose(0, 1, 2, 4, 3, 5)
            .reshape(-1)
        )
        unscramble = lambda o: (
            o.reshape(d2, d3, d0 // 8, d1 // 128, 8, 128)
            .transpose(0, 1, 2, 4, 3, 5)
            .reshape(d2, d3, d0, d1)
            .transpose(2, 3, 0, 1)
        )
    else:
        flat = x.reshape(-1)
        unscramble = lambda o: o.reshape(x.shape)
    if n_pad != n:
        flat = jnp.pad(flat, (0, n_pad - n))

    mesh = plsc.VectorSubcoreMesh(core_axis_name="c", subcore_axis_name="s")
    body = functools.partial(_quantize_body, pattern, K)
    scratch = (
        [pltpu.VMEM((sz,), jnp.float32) for sz in pattern]
        + [pltpu.VMEM((sz,), jnp.float32) for sz in pattern]
        + [pltpu.VMEM((K,), jnp.float32)]
        + [pltpu.SemaphoreType.DMA] * (2 * len(pattern))
    )
    out = pl.kernel(
        body,
        out_type=jax.ShapeDtypeStruct((n_pad,), jnp.float32),
        mesh=mesh,
        scratch_types=scratch,
        compiler_params=pltpu.CompilerParams(needs_layout_passes=False),
    )(flat, center)
    return unscramble(out[:n])
